# Initial kernel scaffold; baseline (speedup 1.0000x reference)
import functools

import jax
import jax.numpy as jnp
from jax import lax
from jax.experimental import pallas as pl
from jax.experimental.pallas import tpu as pltpu
from jax.experimental.pallas import tpu_sc as plsc

NUM_DAYS = 512
EMB = 128
NC, NS = 2, 16
NW = NC * NS
CHUNK = 128
CNT_W = 16


def _sc_partial_sums(news_ids, category_ids, day_ids, news_table, cat_table,
                     zeros_z, zeros_c, ones_c):
    n = news_ids.shape[0]
    per_w = n // NW
    n_chunks = per_w // CHUNK
    rows_per_sub = NUM_DAYS // NS
    mesh = plsc.VectorSubcoreMesh(core_axis_name="c", subcore_axis_name="s")

    @functools.partial(
        pl.kernel,
        out_type=(
            jax.ShapeDtypeStruct((NC, NUM_DAYS, EMB), jnp.float32),
            jax.ShapeDtypeStruct((NC, NUM_DAYS, CNT_W), jnp.float32),
        ),
        mesh=mesh,
        scratch_types=[
            pltpu.VMEM((CHUNK,), jnp.int32),
            pltpu.VMEM((CHUNK,), jnp.int32),
            pltpu.VMEM((CHUNK,), jnp.int32),
            pltpu.VMEM((CHUNK, EMB), jnp.float32),
            pltpu.VMEM((CHUNK, CNT_W), jnp.float32),
            pltpu.VMEM_SHARED((NUM_DAYS, EMB), jnp.float32),
            pltpu.VMEM_SHARED((NUM_DAYS, CNT_W), jnp.float32),
        ],
    )
    def k(nid_hbm, cid_hbm, did_hbm, news_hbm, cat_hbm, z0_hbm, c0_hbm,
          ones_hbm, zp_hbm, cp_hbm,
          nidx_v, cidx_v, didx_v, rows_v, ones_v, zacc_s, cacc_s):
        core = lax.axis_index("c")
        sid = lax.axis_index("s")
        wid = sid * NC + core
        my_rows = pl.ds(sid * rows_per_sub, rows_per_sub)

        pltpu.sync_copy(z0_hbm.at[my_rows], zacc_s.at[my_rows])
        pltpu.sync_copy(c0_hbm.at[my_rows], cacc_s.at[my_rows])
        pltpu.sync_copy(ones_hbm, ones_v)
        plsc.subcore_barrier()

        base_w = wid * per_w

        @pl.loop(0, n_chunks)
        def _(i):
            base = base_w + i * CHUNK
            pltpu.sync_copy(nid_hbm.at[pl.ds(base, CHUNK)], nidx_v)
            pltpu.sync_copy(cid_hbm.at[pl.ds(base, CHUNK)], cidx_v)
            pltpu.sync_copy(did_hbm.at[pl.ds(base, CHUNK)], didx_v)
            pltpu.sync_copy(news_hbm.at[nidx_v], rows_v)
            pltpu.sync_copy(rows_v, zacc_s.at[didx_v], add=True)
            pltpu.sync_copy(cat_hbm.at[cidx_v], rows_v)
            pltpu.sync_copy(rows_v, zacc_s.at[didx_v], add=True)
            pltpu.sync_copy(ones_v, cacc_s.at[didx_v], add=True)

        plsc.subcore_barrier()
        pltpu.sync_copy(zacc_s.at[my_rows], zp_hbm.at[core].at[my_rows])
        pltpu.sync_copy(cacc_s.at[my_rows], cp_hbm.at[core].at[my_rows])

    return k(news_ids, category_ids, day_ids, news_table, cat_table,
             zeros_z, zeros_c, ones_c)


def _tc_combine(zp, cp):
    def body(zp_ref, cp_ref, out_ref):
        z = zp_ref[0] + zp_ref[1]
        c = cp_ref[0, :, 0:1] + cp_ref[1, :, 0:1]
        out_ref[...] = z / jnp.maximum(c, 1.0)

    return pl.pallas_call(
        body,
        out_shape=jax.ShapeDtypeStruct((NUM_DAYS, EMB), jnp.float32),
    )(zp, cp)


def kernel(news_ids, category_ids, day_ids, delta_days, news_table, cat_table):
    nid = news_ids.astype(jnp.int32)
    cid = category_ids.astype(jnp.int32)
    did = day_ids.astype(jnp.int32)
    zeros_z = jnp.zeros((NUM_DAYS, EMB), jnp.float32)
    zeros_c = jnp.zeros((NUM_DAYS, CNT_W), jnp.float32)
    ones_c = jnp.ones((CHUNK, CNT_W), jnp.float32)
    zp, cp = _sc_partial_sums(nid, cid, did, news_table, cat_table,
                              zeros_z, zeros_c, ones_c)
    Z = _tc_combine(zp, cp)
    return (Z, delta_days.astype(jnp.float32))


# speedup vs baseline: 3.9205x; 3.9205x over previous
import functools

import jax
import jax.numpy as jnp
from jax import lax
from jax.experimental import pallas as pl
from jax.experimental.pallas import tpu as pltpu
from jax.experimental.pallas import tpu_sc as plsc

NUM_DAYS = 512
EMB = 128
NC, NS = 2, 16
NW = NC * NS
CHUNK = 128
CNT_W = 128


def _sc_partial_sums(news_ids, category_ids, day_ids, news_table, cat_table,
                     zeros_z, zeros_c, ones_c):
    n = news_ids.shape[0]
    per_w = n // NW
    n_chunks = per_w // CHUNK
    rows_per_sub = NUM_DAYS // NS
    mesh = plsc.VectorSubcoreMesh(core_axis_name="c", subcore_axis_name="s")

    @functools.partial(
        pl.kernel,
        out_type=(
            jax.ShapeDtypeStruct((NC, NUM_DAYS, EMB), jnp.float32),
            jax.ShapeDtypeStruct((NC, NUM_DAYS, CNT_W), jnp.float32),
        ),
        mesh=mesh,
        scratch_types=[
            pltpu.VMEM((CHUNK,), jnp.int32),
            pltpu.VMEM((CHUNK,), jnp.int32),
            pltpu.VMEM((CHUNK,), jnp.int32),
            pltpu.VMEM((CHUNK, EMB), jnp.float32),
            pltpu.VMEM((CHUNK, CNT_W), jnp.float32),
            pltpu.VMEM_SHARED((NUM_DAYS, EMB), jnp.float32),
            pltpu.VMEM_SHARED((NUM_DAYS, CNT_W), jnp.float32),
        ],
    )
    def k(nid_hbm, cid_hbm, did_hbm, news_hbm, cat_hbm, z0_hbm, c0_hbm,
          ones_hbm, zp_hbm, cp_hbm,
          nidx_v, cidx_v, didx_v, rows_v, ones_v, zacc_s, cacc_s):
        core = lax.axis_index("c")
        sid = lax.axis_index("s")
        wid = sid * NC + core
        my_rows = pl.ds(sid * rows_per_sub, rows_per_sub)

        pltpu.sync_copy(z0_hbm.at[my_rows], zacc_s.at[my_rows])
        pltpu.sync_copy(c0_hbm.at[my_rows], cacc_s.at[my_rows])
        pltpu.sync_copy(ones_hbm, ones_v)
        plsc.subcore_barrier()

        base_w = wid * per_w

        @pl.loop(0, n_chunks)
        def _(i):
            base = base_w + i * CHUNK
            pltpu.sync_copy(nid_hbm.at[pl.ds(base, CHUNK)], nidx_v)
            pltpu.sync_copy(cid_hbm.at[pl.ds(base, CHUNK)], cidx_v)
            pltpu.sync_copy(did_hbm.at[pl.ds(base, CHUNK)], didx_v)
            pltpu.sync_copy(news_hbm.at[nidx_v], rows_v)
            pltpu.sync_copy(rows_v, zacc_s.at[didx_v], add=True)
            pltpu.sync_copy(cat_hbm.at[cidx_v], rows_v)
            pltpu.sync_copy(rows_v, zacc_s.at[didx_v], add=True)
            pltpu.sync_copy(ones_v, cacc_s.at[didx_v], add=True)

        plsc.subcore_barrier()
        pltpu.sync_copy(zacc_s.at[my_rows], zp_hbm.at[core].at[my_rows])
        pltpu.sync_copy(cacc_s.at[my_rows], cp_hbm.at[core].at[my_rows])

    return k(news_ids, category_ids, day_ids, news_table, cat_table,
             zeros_z, zeros_c, ones_c)


def _tc_combine(zp, cp):
    def body(zp_ref, cp_ref, out_ref):
        z = zp_ref[0] + zp_ref[1]
        c = cp_ref[0, :, 0:1] + cp_ref[1, :, 0:1]
        out_ref[...] = z / jnp.maximum(c, 1.0)

    return pl.pallas_call(
        body,
        out_shape=jax.ShapeDtypeStruct((NUM_DAYS, EMB), jnp.float32),
    )(zp, cp)


def kernel(news_ids, category_ids, day_ids, delta_days, news_table, cat_table):
    nid = news_ids.astype(jnp.int32)
    cid = category_ids.astype(jnp.int32)
    did = day_ids.astype(jnp.int32)
    zeros_z = jnp.zeros((NUM_DAYS, EMB), jnp.float32)
    zeros_c = jnp.zeros((NUM_DAYS, CNT_W), jnp.float32)
    ones_c = jnp.ones((CHUNK, CNT_W), jnp.float32)
    zp, cp = _sc_partial_sums(nid, cid, did, news_table, cat_table,
                              zeros_z, zeros_c, ones_c)
    Z = _tc_combine(zp, cp)
    return (Z, delta_days.astype(jnp.float32))


# speedup vs baseline: 5.0661x; 1.2922x over previous
import functools

import jax
import jax.numpy as jnp
from jax import lax
from jax.experimental import pallas as pl
from jax.experimental.pallas import tpu as pltpu
from jax.experimental.pallas import tpu_sc as plsc

NUM_DAYS = 512
EMB = 128
NC, NS = 2, 16
NW = NC * NS
CHUNK = 128
CNT_W = 128
NSLOT = 2


def _sc_partial_sums(ids_packed, news_table, cat_table, zeros_z, ones_c):
    n_chunks_total = ids_packed.shape[0]
    n_chunks = n_chunks_total // NW
    rows_per_sub = NUM_DAYS // NS
    cat_rows = cat_table.shape[0]
    mesh = plsc.VectorSubcoreMesh(core_axis_name="c", subcore_axis_name="s")

    step = -(-cat_rows // NS)
    step += (-step) % 8
    cat_starts, cat_sizes = [], []
    for sid_py in range(NS):
        start = sid_py * step
        sz = max(0, min(step, cat_rows - start))
        cat_starts.append(start)
        cat_sizes.append(sz)

    @functools.partial(
        pl.kernel,
        out_type=(
            jax.ShapeDtypeStruct((NC, NUM_DAYS, EMB), jnp.float32),
            jax.ShapeDtypeStruct((NC, NUM_DAYS, CNT_W), jnp.float32),
        ),
        mesh=mesh,
        scratch_types=[
            pltpu.VMEM((NSLOT, 3, CHUNK), jnp.int32),
            pltpu.VMEM((NSLOT, CHUNK, EMB), jnp.float32),
            pltpu.VMEM((NSLOT, CHUNK, EMB), jnp.float32),
            pltpu.VMEM((CHUNK, CNT_W), jnp.float32),
            pltpu.VMEM_SHARED((NUM_DAYS, EMB), jnp.float32),
            pltpu.VMEM_SHARED((NUM_DAYS, CNT_W), jnp.float32),
        ] + [pltpu.SemaphoreType.DMA] * (2 * NSLOT),
    )
    def k(ids_hbm, news_hbm, cat_hbm, z0_hbm, ones_hbm, zp_hbm, cp_hbm,
          idx_v, bufn_v, bufc_v, ones_v, zacc_s, cacc_s, *sems):
        gsem = sems[:NSLOT]
        ssem = sems[NSLOT:]
        core = lax.axis_index("c")
        sid = lax.axis_index("s")
        wid = sid * NC + core
        my_rows = pl.ds(sid * rows_per_sub, rows_per_sub)

        pltpu.sync_copy(z0_hbm.at[my_rows], zacc_s.at[my_rows])
        pltpu.sync_copy(z0_hbm.at[my_rows], cacc_s.at[my_rows])
        pltpu.sync_copy(ones_hbm, ones_v)
        plsc.subcore_barrier()

        chunk0 = wid * n_chunks

        gd = [None] * NSLOT
        sd = [None] * NSLOT

        def fire_scatters(slot):
            for d in gd[slot]:
                d.wait()
            day_idx = idx_v.at[slot].at[2]
            sd[slot] = [
                pltpu.async_copy(bufn_v.at[slot], zacc_s.at[day_idx],
                                 ssem[slot], add=True),
                pltpu.async_copy(bufc_v.at[slot], zacc_s.at[day_idx],
                                 ssem[slot], add=True),
                pltpu.async_copy(ones_v, cacc_s.at[day_idx],
                                 ssem[slot], add=True),
            ]

        for i in range(n_chunks):
            s = i % NSLOT
            if sd[s] is not None:
                for d in sd[s]:
                    d.wait()
            pltpu.sync_copy(ids_hbm.at[chunk0 + i], idx_v.at[s])
            gd[s] = [
                pltpu.async_copy(news_hbm.at[idx_v.at[s].at[0]],
                                 bufn_v.at[s], gsem[s]),
                pltpu.async_copy(cat_hbm.at[idx_v.at[s].at[1]],
                                 bufc_v.at[s], gsem[s]),
            ]
            if i >= 1:
                fire_scatters((i - 1) % NSLOT)
        fire_scatters((n_chunks - 1) % NSLOT)
        for slot in range(NSLOT):
            if sd[slot] is not None:
                for d in sd[slot]:
                    d.wait()

        plsc.subcore_barrier()
        pltpu.sync_copy(zacc_s.at[my_rows], zp_hbm.at[core].at[my_rows])
        pltpu.sync_copy(cacc_s.at[my_rows], cp_hbm.at[core].at[my_rows])

    return k(ids_packed, news_table, cat_table, zeros_z, ones_c)


def _tc_combine(zp, cp):
    def body(zp_ref, cp_ref, out_ref):
        z = zp_ref[0] + zp_ref[1]
        c = cp_ref[0, :, 0:1] + cp_ref[1, :, 0:1]
        out_ref[...] = z / jnp.maximum(c, 1.0)

    return pl.pallas_call(
        body,
        out_shape=jax.ShapeDtypeStruct((NUM_DAYS, EMB), jnp.float32),
    )(zp, cp)


def kernel(news_ids, category_ids, day_ids, delta_days, news_table, cat_table):
    n = news_ids.shape[0]
    n_chunks_total = n // CHUNK
    ids_packed = jnp.stack(
        [news_ids.astype(jnp.int32).reshape(n_chunks_total, CHUNK),
         category_ids.astype(jnp.int32).reshape(n_chunks_total, CHUNK),
         day_ids.astype(jnp.int32).reshape(n_chunks_total, CHUNK)],
        axis=1)
    zeros_z = jnp.zeros((NUM_DAYS, EMB), jnp.float32)
    ones_c = jnp.ones((CHUNK, CNT_W), jnp.float32)
    zp, cp = _sc_partial_sums(ids_packed, news_table, cat_table,
                              zeros_z, ones_c)
    Z = _tc_combine(zp, cp)
    return (Z, delta_days.astype(jnp.float32))


# speedup vs baseline: 5.1494x; 1.0164x over previous
import functools

import jax
import jax.numpy as jnp
from jax import lax
from jax.experimental import pallas as pl
from jax.experimental.pallas import tpu as pltpu
from jax.experimental.pallas import tpu_sc as plsc

NUM_DAYS = 512
EMB = 128
NC, NS = 2, 16
NW = NC * NS
CHUNK = 128
CNT_W = 128
NSLOT = 3


def _sc_partial_sums(ids_packed, news_table, cat_table, zeros_z, ones_c):
    n_chunks_total = ids_packed.shape[0]
    n_chunks = n_chunks_total // NW
    rows_per_sub = NUM_DAYS // NS
    cat_rows = cat_table.shape[0]
    mesh = plsc.VectorSubcoreMesh(core_axis_name="c", subcore_axis_name="s")

    step = -(-cat_rows // NS)
    step += (-step) % 8
    cat_starts, cat_sizes = [], []
    for sid_py in range(NS):
        start = sid_py * step
        sz = max(0, min(step, cat_rows - start))
        cat_starts.append(start)
        cat_sizes.append(sz)

    @functools.partial(
        pl.kernel,
        out_type=(
            jax.ShapeDtypeStruct((NC, NUM_DAYS, EMB), jnp.float32),
            jax.ShapeDtypeStruct((NC, NUM_DAYS, CNT_W), jnp.float32),
        ),
        mesh=mesh,
        scratch_types=[
            pltpu.VMEM((n_chunks, 3, CHUNK), jnp.int32),
            pltpu.VMEM((NSLOT, CHUNK, EMB), jnp.float32),
            pltpu.VMEM((NSLOT, CHUNK, EMB), jnp.float32),
            pltpu.VMEM((CHUNK, CNT_W), jnp.float32),
            pltpu.VMEM_SHARED((NUM_DAYS, EMB), jnp.float32),
            pltpu.VMEM_SHARED((NUM_DAYS, CNT_W), jnp.float32),
        ] + [pltpu.SemaphoreType.DMA] * (2 * NSLOT),
    )
    def k(ids_hbm, news_hbm, cat_hbm, z0_hbm, ones_hbm, zp_hbm, cp_hbm,
          idx_v, bufn_v, bufc_v, ones_v, zacc_s, cacc_s, *sems):
        gsem = sems[:NSLOT]
        ssem = sems[NSLOT:]
        core = lax.axis_index("c")
        sid = lax.axis_index("s")
        wid = sid * NC + core
        my_rows = pl.ds(sid * rows_per_sub, rows_per_sub)

        pltpu.sync_copy(z0_hbm.at[my_rows], zacc_s.at[my_rows])
        pltpu.sync_copy(z0_hbm.at[my_rows], cacc_s.at[my_rows])
        pltpu.sync_copy(ones_hbm, ones_v)
        pltpu.sync_copy(ids_hbm.at[pl.ds(wid * n_chunks, n_chunks)], idx_v)
        plsc.subcore_barrier()

        gd = [None] * NSLOT
        sd = [None] * NSLOT

        def fire_scatters(i):
            slot = i % NSLOT
            for d in gd[slot]:
                d.wait()
            day_idx = idx_v.at[i].at[2]
            sd[slot] = [
                pltpu.async_copy(bufn_v.at[slot], zacc_s.at[day_idx],
                                 ssem[slot], add=True),
                pltpu.async_copy(bufc_v.at[slot], zacc_s.at[day_idx],
                                 ssem[slot], add=True),
                pltpu.async_copy(ones_v, cacc_s.at[day_idx],
                                 ssem[slot], add=True),
            ]

        for i in range(n_chunks):
            s = i % NSLOT
            if sd[s] is not None:
                for d in sd[s]:
                    d.wait()
                sd[s] = None
            gd[s] = [
                pltpu.async_copy(news_hbm.at[idx_v.at[i].at[0]],
                                 bufn_v.at[s], gsem[s]),
                pltpu.async_copy(cat_hbm.at[idx_v.at[i].at[1]],
                                 bufc_v.at[s], gsem[s]),
            ]
            if i >= 1:
                fire_scatters(i - 1)
        fire_scatters(n_chunks - 1)
        for slot in range(NSLOT):
            if sd[slot] is not None:
                for d in sd[slot]:
                    d.wait()

        plsc.subcore_barrier()
        pltpu.sync_copy(zacc_s.at[my_rows], zp_hbm.at[core].at[my_rows])
        pltpu.sync_copy(cacc_s.at[my_rows], cp_hbm.at[core].at[my_rows])

    return k(ids_packed, news_table, cat_table, zeros_z, ones_c)


def _tc_combine(zp, cp):
    def body(zp_ref, cp_ref, out_ref):
        z = zp_ref[0] + zp_ref[1]
        c = cp_ref[0, :, 0:1] + cp_ref[1, :, 0:1]
        out_ref[...] = z / jnp.maximum(c, 1.0)

    return pl.pallas_call(
        body,
        out_shape=jax.ShapeDtypeStruct((NUM_DAYS, EMB), jnp.float32),
    )(zp, cp)


def kernel(news_ids, category_ids, day_ids, delta_days, news_table, cat_table):
    n = news_ids.shape[0]
    n_chunks_total = n // CHUNK
    ids_packed = jnp.stack(
        [news_ids.astype(jnp.int32).reshape(n_chunks_total, CHUNK),
         category_ids.astype(jnp.int32).reshape(n_chunks_total, CHUNK),
         day_ids.astype(jnp.int32).reshape(n_chunks_total, CHUNK)],
        axis=1)
    zeros_z = jnp.zeros((NUM_DAYS, EMB), jnp.float32)
    ones_c = jnp.ones((CHUNK, CNT_W), jnp.float32)
    zp, cp = _sc_partial_sums(ids_packed, news_table, cat_table,
                              zeros_z, ones_c)
    Z = _tc_combine(zp, cp)
    return (Z, delta_days.astype(jnp.float32))
